# baseline (speedup 1.0000x reference)
import jax
import jax.numpy as jnp
from jax import lax
from jax.experimental import pallas as pl
from jax.experimental.pallas import tpu as pltpu
from jax.experimental.pallas import tpu_sc as plsc

VOCAB = 1000000
EMBED = 64
B = 4096
L = 200

NUM_IDX = B * L
NC, NS = 2, 16
NW = NC * NS
PER_W = NUM_IDX // NW
WIN = 128
NWIN = PER_W // WIN
NBUF = 2
PAIR_LANES = 2 * EMBED
VREG = 16


def _sc_gather(table_pairs, idx_half, half_off):
    mesh = plsc.VectorSubcoreMesh(core_axis_name="c", subcore_axis_name="s")

    scratch = [
        pltpu.VMEM((PER_W,), jnp.int32),
        pltpu.VMEM((PER_W,), jnp.int32),
    ]
    scratch += [pltpu.VMEM((WIN, PAIR_LANES), jnp.float32) for _ in range(NBUF)]
    scratch += [pltpu.VMEM((WIN, EMBED), jnp.float32) for _ in range(NBUF)]
    scratch += [pltpu.SemaphoreType.DMA for _ in range(NBUF)]

    @pl.kernel(
        out_type=jax.ShapeDtypeStruct((NUM_IDX, EMBED), jnp.float32),
        mesh=mesh,
        scratch_types=scratch,
    )
    def k(table_hbm, ih_hbm, ho_hbm, out_hbm, ih_v, ho_v, *rest):
        wbuf = rest[:NBUF]
        obuf = rest[NBUF:2 * NBUF]
        gsem = rest[2 * NBUF:3 * NBUF]

        wid = lax.axis_index("s") * NC + lax.axis_index("c")
        base = wid * PER_W
        pltpu.sync_copy(ih_hbm.at[pl.ds(base, PER_W)], ih_v)
        pltpu.sync_copy(ho_hbm.at[pl.ds(base, PER_W)], ho_v)

        def compact(j, w):
            @pl.loop(0, WIN, step=VREG)
            def _(r0):
                offs = ho_v[pl.ds(w * WIN + r0, VREG)]
                for l in range(VREG):
                    off = offs[l]
                    for c in range(0, EMBED, VREG):
                        obuf[j][r0 + l, pl.ds(c, VREG)] = wbuf[j][
                            r0 + l, pl.ds(off + c, VREG)
                        ]

        @pl.loop(0, NWIN, step=NBUF)
        def _(g):
            handles = []
            for j in range(NBUF):
                h = pltpu.async_copy(
                    table_hbm.at[ih_v.at[pl.ds((g + j) * WIN, WIN)]],
                    wbuf[j],
                    gsem[j],
                )
                handles.append(h)
            for j in range(NBUF):
                handles[j].wait()
                compact(j, g + j)
                pltpu.sync_copy(
                    obuf[j], out_hbm.at[pl.ds(base + (g + j) * WIN, WIN)]
                )

    return k(table_pairs, idx_half, half_off)


def kernel(cat_seq, embed_cat):
    idx = cat_seq.reshape(NUM_IDX).astype(jnp.int32)
    idx_half = idx >> 1
    half_off = (idx & 1) * EMBED
    table_pairs = embed_cat.reshape(VOCAB // 2, PAIR_LANES)
    out = _sc_gather(table_pairs, idx_half, half_off)
    return out.reshape(B, L, EMBED)
